# baseline (speedup 1.0000x reference)
import functools

import jax
import jax.numpy as jnp
from jax import lax
from jax.experimental import pallas as pl
from jax.experimental.pallas import tpu as pltpu
from jax.experimental.pallas import tpu_sc as plsc

B = 16384
D = 64
NW = 32
BPW = B // NW
C = 64
NCHUNK = BPW // C
L = 16
NG = C // L
N_REL = 500
N_TIME = 366
HALF = 500000

_mesh = plsc.VectorSubcoreMesh(core_axis_name="c", subcore_axis_name="s")


@functools.partial(
    pl.kernel,
    mesh=_mesh,
    compiler_params=pltpu.CompilerParams(
        needs_layout_passes=False, use_tc_tiling_on_sc=False),
    out_type=(
        jax.ShapeDtypeStruct((B,), jnp.float32),
        jax.ShapeDtypeStruct((B,), jnp.float32),
    ),
    scratch_types=[
        pltpu.VMEM((BPW,), jnp.int32),
        pltpu.VMEM((BPW,), jnp.int32),
        pltpu.VMEM((BPW,), jnp.int32),
        pltpu.VMEM((BPW,), jnp.int32),
        pltpu.VMEM((BPW,), jnp.int32),
        pltpu.VMEM((BPW,), jnp.int32),
        pltpu.VMEM((BPW,), jnp.int32),
        pltpu.VMEM((C,), jnp.int32),
        pltpu.VMEM((C,), jnp.int32),
        pltpu.VMEM((C,), jnp.int32),
        pltpu.VMEM((C,), jnp.int32),
        pltpu.VMEM((C,), jnp.int32),
        pltpu.VMEM((C,), jnp.int32),
        pltpu.VMEM((C,), jnp.int32),
        pltpu.VMEM((C,), jnp.int32),
        pltpu.VMEM((C, D), jnp.float32),
        pltpu.VMEM((C, D), jnp.float32),
        pltpu.VMEM((C, D), jnp.float32),
        pltpu.VMEM((C, D), jnp.float32),
        pltpu.VMEM((C, D), jnp.float32),
        pltpu.VMEM((C, D), jnp.float32),
        pltpu.VMEM((C, D), jnp.float32),
        pltpu.VMEM((C, D), jnp.float32),
        pltpu.VMEM((N_REL * D,), jnp.float32),
        pltpu.VMEM((N_TIME * D,), jnp.float32),
        pltpu.VMEM((BPW,), jnp.float32),
        pltpu.VMEM((BPW,), jnp.float32),
        pltpu.SemaphoreType.DMA,
    ],
)
def _score_kernel(h_hbm, t_hbm, nh_hbm, nt_hbm, r_hbm, st_hbm, et_hbm,
                  etop_hbm, ebot_hbm, rel_hbm, time_hbm, pos_hbm, neg_hbm,
                  hi_v, ti_v, nhi_v, nti_v, ri_v, si_v, ei_v,
                  ht_v, hb_v, tt_v, tb_v, nht_v, nhb_v, ntt_v, ntb_v,
                  h0_v, h1_v, t0_v, t1_v, nh0_v, nh1_v, nt0_v, nt1_v,
                  rel_c, time_c, pos_v, neg_v, sem):
    wid = lax.axis_index("s") * 2 + lax.axis_index("c")
    wb = wid * BPW
    pltpu.sync_copy(h_hbm.at[pl.ds(wb, BPW)], hi_v)
    pltpu.sync_copy(t_hbm.at[pl.ds(wb, BPW)], ti_v)
    pltpu.sync_copy(nh_hbm.at[pl.ds(wb, BPW)], nhi_v)
    pltpu.sync_copy(nt_hbm.at[pl.ds(wb, BPW)], nti_v)
    pltpu.sync_copy(r_hbm.at[pl.ds(wb, BPW)], ri_v)
    pltpu.sync_copy(st_hbm.at[pl.ds(wb, BPW)], si_v)
    pltpu.sync_copy(et_hbm.at[pl.ds(wb, BPW)], ei_v)
    pltpu.sync_copy(rel_hbm, rel_c)
    pltpu.sync_copy(time_hbm, time_c)

    def chunk_body(c, _):
        base = c * C

        @plsc.parallel_loop(0, C, step=L)
        def mk_idx(k):
            spread = lax.iota(jnp.int32, L) + k
            for src, top, bot in ((hi_v, ht_v, hb_v), (ti_v, tt_v, tb_v),
                                  (nhi_v, nht_v, nhb_v), (nti_v, ntt_v, ntb_v)):
                iv = src[pl.ds(base + k, L)]
                top[pl.ds(k, L)] = jnp.minimum(iv, HALF - 1)
                bot[pl.ds(k, L)] = jnp.where(iv >= HALF, iv - HALF, spread)

        cps = [
            pltpu.async_copy(etop_hbm.at[ht_v], h0_v, sem),
            pltpu.async_copy(ebot_hbm.at[hb_v], h1_v, sem),
            pltpu.async_copy(etop_hbm.at[tt_v], t0_v, sem),
            pltpu.async_copy(ebot_hbm.at[tb_v], t1_v, sem),
            pltpu.async_copy(etop_hbm.at[nht_v], nh0_v, sem),
            pltpu.async_copy(ebot_hbm.at[nhb_v], nh1_v, sem),
            pltpu.async_copy(etop_hbm.at[ntt_v], nt0_v, sem),
            pltpu.async_copy(ebot_hbm.at[ntb_v], nt1_v, sem),
        ]
        for cp in cps:
            cp.wait()

        @plsc.parallel_loop(0, NG)
        def grp(g):
            sl = pl.ds(base + g * L, L)
            slot = lax.iota(jnp.int32, L) + g * L
            htop = hi_v[sl] < HALF
            ttop = ti_v[sl] < HALF
            nhtop = nhi_v[sl] < HALF
            nttop = nti_v[sl] < HALF
            rbase = ri_v[sl] * D
            sbase = si_v[sl] * D
            ebase = ei_v[sl] * D
            accp = jnp.zeros((L,), jnp.float32)
            accn = jnp.zeros((L,), jnp.float32)
            for d in range(D):
                dv = jnp.full((L,), d, jnp.int32)
                hv = jnp.where(htop, plsc.load_gather(h0_v, [slot, dv]),
                               plsc.load_gather(h1_v, [slot, dv]))
                tv = jnp.where(ttop, plsc.load_gather(t0_v, [slot, dv]),
                               plsc.load_gather(t1_v, [slot, dv]))
                nhv = jnp.where(nhtop, plsc.load_gather(nh0_v, [slot, dv]),
                                plsc.load_gather(nh1_v, [slot, dv]))
                ntv = jnp.where(nttop, plsc.load_gather(nt0_v, [slot, dv]),
                                plsc.load_gather(nt1_v, [slot, dv]))
                rv = plsc.load_gather(rel_c, [rbase + d])
                tsv = plsc.load_gather(time_c, [sbase + d])
                tev = plsc.load_gather(time_c, [ebase + d])
                trans = rv + 0.5 * (tsv + tev)
                accp = accp + jnp.abs(hv + trans - tv)
                accn = accn + jnp.abs(nhv + trans - ntv)
            pos_v[sl] = -accp
            neg_v[sl] = -accn

        return 0

    lax.fori_loop(0, NCHUNK, chunk_body, 0)

    pltpu.sync_copy(pos_v, pos_hbm.at[pl.ds(wb, BPW)])
    pltpu.sync_copy(neg_v, neg_hbm.at[pl.ds(wb, BPW)])


def kernel(heads, tails, relations, start_time, end_time,
           negative_heads, negative_tails, ent_emb, rel_emb, time_emb):
    pos, neg = _score_kernel(
        heads.astype(jnp.int32), tails.astype(jnp.int32),
        negative_heads.astype(jnp.int32), negative_tails.astype(jnp.int32),
        relations.astype(jnp.int32), start_time.astype(jnp.int32),
        end_time.astype(jnp.int32),
        ent_emb[:HALF], ent_emb[HALF:],
        rel_emb.reshape(-1), time_emb.reshape(-1))
    return pos, neg


# speedup vs baseline: 3.1365x; 3.1365x over previous
import functools

import jax
import jax.numpy as jnp
from jax import lax
from jax.experimental import pallas as pl
from jax.experimental.pallas import tpu as pltpu
from jax.experimental.pallas import tpu_sc as plsc

B = 16384
D = 64
NW = 32
BPW = B // NW
C = 16
NCHUNK = BPW // C
L = 16
N_REL = 500
N_TIME = 366

_mesh = plsc.VectorSubcoreMesh(core_axis_name="c", subcore_axis_name="s")

_row_bufs = [pltpu.VMEM((C, D), jnp.float32)] * 8


@functools.partial(
    pl.kernel,
    mesh=_mesh,
    compiler_params=pltpu.CompilerParams(needs_layout_passes=False),
    out_type=(
        jax.ShapeDtypeStruct((B,), jnp.float32),
        jax.ShapeDtypeStruct((B,), jnp.float32),
    ),
    scratch_types=[
        pltpu.VMEM((BPW,), jnp.int32),
        pltpu.VMEM((BPW,), jnp.int32),
        pltpu.VMEM((BPW,), jnp.int32),
        pltpu.VMEM((BPW,), jnp.int32),
        pltpu.VMEM((BPW,), jnp.int32),
        pltpu.VMEM((BPW,), jnp.int32),
        pltpu.VMEM((BPW,), jnp.int32),
        *_row_bufs,
        pltpu.VMEM((N_REL * D,), jnp.float32),
        pltpu.VMEM((N_TIME * D,), jnp.float32),
        pltpu.VMEM((BPW,), jnp.float32),
        pltpu.VMEM((BPW,), jnp.float32),
        pltpu.SemaphoreType.DMA,
        pltpu.SemaphoreType.DMA,
    ],
)
def _score_kernel(h_hbm, t_hbm, nh_hbm, nt_hbm, r_hbm, st_hbm, et_hbm,
                  ent_hbm, rel_hbm, time_hbm, pos_hbm, neg_hbm,
                  hi_v, ti_v, nhi_v, nti_v, ri_v, si_v, ei_v,
                  h0_v, t0_v, nh0_v, nt0_v, h1_v, t1_v, nh1_v, nt1_v,
                  rel_c, time_c, pos_v, neg_v, sem0, sem1):
    wid = lax.axis_index("s") * 2 + lax.axis_index("c")
    wb = wid * BPW
    pltpu.sync_copy(h_hbm.at[pl.ds(wb, BPW)], hi_v)
    pltpu.sync_copy(t_hbm.at[pl.ds(wb, BPW)], ti_v)
    pltpu.sync_copy(nh_hbm.at[pl.ds(wb, BPW)], nhi_v)
    pltpu.sync_copy(nt_hbm.at[pl.ds(wb, BPW)], nti_v)
    pltpu.sync_copy(r_hbm.at[pl.ds(wb, BPW)], ri_v)
    pltpu.sync_copy(st_hbm.at[pl.ds(wb, BPW)], si_v)
    pltpu.sync_copy(et_hbm.at[pl.ds(wb, BPW)], ei_v)
    pltpu.sync_copy(rel_hbm, rel_c)
    pltpu.sync_copy(time_hbm, time_c)

    sets = ((h0_v, t0_v, nh0_v, nt0_v, sem0),
            (h1_v, t1_v, nh1_v, nt1_v, sem1))

    def fire(c, which):
        h_v, t_v, nh_v, nt_v, sem = sets[which]
        base = c * C
        hvec = hi_v[pl.ds(base, L)]
        tvec = ti_v[pl.ds(base, L)]
        nhvec = nhi_v[pl.ds(base, L)]
        ntvec = nti_v[pl.ds(base, L)]
        cps = []
        for l in range(L):
            cps.append(pltpu.async_copy(ent_hbm.at[hvec[l]], h_v.at[l], sem))
            cps.append(pltpu.async_copy(ent_hbm.at[tvec[l]], t_v.at[l], sem))
            cps.append(pltpu.async_copy(ent_hbm.at[nhvec[l]], nh_v.at[l], sem))
            cps.append(pltpu.async_copy(ent_hbm.at[ntvec[l]], nt_v.at[l], sem))
        return cps

    def drain(which):
        h_v, t_v, nh_v, nt_v, sem = sets[which]
        dummy = pltpu.make_async_copy(ent_hbm.at[0], h_v.at[0], sem)
        for _ in range(4 * L):
            dummy.wait()

    def compute(c, which):
        h_v, t_v, nh_v, nt_v, _ = sets[which]
        base = c * C
        sl = pl.ds(base, L)
        slot = lax.iota(jnp.int32, L)
        rbase = ri_v[sl] * D
        sbase = si_v[sl] * D
        ebase = ei_v[sl] * D
        accp = jnp.zeros((L,), jnp.float32)
        accn = jnp.zeros((L,), jnp.float32)
        for d in range(D):
            dv = jnp.full((L,), d, jnp.int32)
            hv = plsc.load_gather(h_v, [slot, dv])
            tv = plsc.load_gather(t_v, [slot, dv])
            nhv = plsc.load_gather(nh_v, [slot, dv])
            ntv = plsc.load_gather(nt_v, [slot, dv])
            rv = plsc.load_gather(rel_c, [rbase + d])
            tsv = plsc.load_gather(time_c, [sbase + d])
            tev = plsc.load_gather(time_c, [ebase + d])
            trans = rv + 0.5 * (tsv + tev)
            accp = accp + jnp.abs(hv + trans - tv)
            accn = accn + jnp.abs(nhv + trans - ntv)
        pos_v[sl] = -accp
        neg_v[sl] = -accn

    fire(0, 0)
    fire(1, 1)

    def pair_body(cc, _):
        c0 = cc * 2
        drain(0)
        compute(c0, 0)

        @pl.when(c0 + 2 < NCHUNK)
        def _():
            fire(c0 + 2, 0)

        drain(1)
        compute(c0 + 1, 1)

        @pl.when(c0 + 3 < NCHUNK)
        def _():
            fire(c0 + 3, 1)

        return 0

    lax.fori_loop(0, NCHUNK // 2, pair_body, 0)

    pltpu.sync_copy(pos_v, pos_hbm.at[pl.ds(wb, BPW)])
    pltpu.sync_copy(neg_v, neg_hbm.at[pl.ds(wb, BPW)])


def kernel(heads, tails, relations, start_time, end_time,
           negative_heads, negative_tails, ent_emb, rel_emb, time_emb):
    pos, neg = _score_kernel(
        heads.astype(jnp.int32), tails.astype(jnp.int32),
        negative_heads.astype(jnp.int32), negative_tails.astype(jnp.int32),
        relations.astype(jnp.int32), start_time.astype(jnp.int32),
        end_time.astype(jnp.int32),
        ent_emb, rel_emb.reshape(-1), time_emb.reshape(-1))
    return pos, neg
